# baseline (speedup 1.0000x reference)
import functools

import jax
import jax.numpy as jnp
from jax.experimental import pallas as pl
from jax.experimental.pallas import tpu as pltpu

_VMEM_BUDGET = (64 * 1024 * 1024 * 3) // 4
_BLOCK = 1024


def _dot_bias(x, w, b):
    acc = jax.lax.dot_general(
        x, w, dimension_numbers=(((1,), (1,)), ((), ())),
        preferred_element_type=jnp.float32)
    return (acc + b.astype(jnp.float32)).astype(x.dtype)


def _manual_kernel(x_hbm, w_ref, b_ref, o_hbm, x_buf, o_buf, w_bf, in_sem,
                   out_sem, *, block, n_steps):
    def dma_in(slot, step):
        pltpu.make_async_copy(
            x_hbm.at[pl.ds(step * block, block), :], x_buf.at[slot],
            in_sem.at[slot]).start()

    def wait_in(slot):
        pltpu.make_async_copy(
            x_hbm.at[pl.ds(0, block), :], x_buf.at[slot],
            in_sem.at[slot]).wait()

    def dma_out(slot, step):
        pltpu.make_async_copy(
            o_buf.at[slot], o_hbm.at[pl.ds(step * block, block), :],
            out_sem.at[slot]).start()

    def wait_out(slot):
        pltpu.make_async_copy(
            o_buf.at[slot], o_hbm.at[pl.ds(0, block), :],
            out_sem.at[slot]).wait()

    dma_in(0, 0)
    w_bf[...] = w_ref[...].astype(jnp.bfloat16)

    def body(step, _):
        cur = jax.lax.rem(step, 2)
        nxt = jax.lax.rem(step + 1, 2)

        @pl.when(step + 1 < n_steps)
        def _():
            dma_in(nxt, step + 1)

        wait_in(cur)

        @pl.when(step >= 2)
        def _():
            wait_out(cur)

        o_buf[cur] = _dot_bias(x_buf[cur], w_bf[...], b_ref[...])
        dma_out(cur, step)
        return ()

    jax.lax.fori_loop(0, n_steps, body, (), unroll=False)
    if n_steps >= 2:
        wait_out((n_steps - 2) % 2)
    wait_out((n_steps - 1) % 2)


def _auto_kernel(x_ref, w_ref, b_ref, o_ref):
    o_ref[...] = _dot_bias(x_ref[...], w_ref[...], b_ref[...])


def _forward_auto(x, weight, b_row):
    n, k = x.shape
    h = weight.shape[0]
    tm = min(2048, n)
    return pl.pallas_call(
        _auto_kernel,
        out_shape=jax.ShapeDtypeStruct((n, h), x.dtype),
        grid=(pl.cdiv(n, tm),),
        in_specs=[
            pl.BlockSpec((tm, k), lambda i: (i, 0)),
            pl.BlockSpec((h, k), lambda i: (0, 0)),
            pl.BlockSpec((1, h), lambda i: (0, 0)),
        ],
        out_specs=pl.BlockSpec((tm, h), lambda i: (i, 0)),
        compiler_params=pltpu.CompilerParams(
            dimension_semantics=("arbitrary",),
            vmem_limit_bytes=_VMEM_BUDGET,
        ),
    )(x, weight, b_row)


def kernel(x, weight, bias):
    n, k = x.shape
    h = weight.shape[0]
    b_row = bias.reshape(1, h)

    if n % _BLOCK != 0:
        return _forward_auto(x, weight, b_row)

    n_steps = n // _BLOCK
    bytes_accessed = (x.size * 4 + weight.size * 4 + h * 4 + n * h * 4)

    return pl.pallas_call(
        functools.partial(_manual_kernel, block=_BLOCK, n_steps=n_steps),
        out_shape=jax.ShapeDtypeStruct((n, h), x.dtype),
        in_specs=[
            pl.BlockSpec(memory_space=pl.ANY),
            pl.BlockSpec(memory_space=pltpu.VMEM),
            pl.BlockSpec(memory_space=pltpu.VMEM),
        ],
        out_specs=pl.BlockSpec(memory_space=pl.ANY),
        scratch_shapes=[
            pltpu.VMEM((2, _BLOCK, k), x.dtype),
            pltpu.VMEM((2, _BLOCK, h), x.dtype),
            pltpu.VMEM((h, k), jnp.bfloat16),
            pltpu.SemaphoreType.DMA((2,)),
            pltpu.SemaphoreType.DMA((2,)),
        ],
        compiler_params=pltpu.CompilerParams(
            vmem_limit_bytes=_VMEM_BUDGET,
        ),
        cost_estimate=pl.CostEstimate(
            flops=2 * n * h * k,
            bytes_accessed=bytes_accessed,
            transcendentals=0),
    )(x, weight, b_row)


# speedup vs baseline: 1.0037x; 1.0037x over previous
import functools

import jax
import jax.numpy as jnp
from jax.experimental import pallas as pl
from jax.experimental.pallas import tpu as pltpu

_VMEM_BUDGET = (64 * 1024 * 1024 * 3) // 4
_BLOCK = 1024


def _dot_bias(x, w, b, out_dtype):
    acc = jax.lax.dot_general(
        x, w, dimension_numbers=(((1,), (1,)), ((), ())),
        preferred_element_type=jnp.float32)
    return (acc + b.astype(jnp.float32)).astype(out_dtype)


def _manual_kernel(x_hbm, w_ref, b_ref, o_hbm, x_buf, o_buf, w_bf, x_bf,
                   in_sem, out_sem, *, block, n_steps):
    def dma_in(slot, step):
        pltpu.make_async_copy(
            x_hbm.at[pl.ds(step * block, block), :], x_buf.at[slot],
            in_sem.at[slot]).start()

    def wait_in(slot):
        pltpu.make_async_copy(
            x_hbm.at[pl.ds(0, block), :], x_buf.at[slot],
            in_sem.at[slot]).wait()

    def dma_out(slot, step):
        pltpu.make_async_copy(
            o_buf.at[slot], o_hbm.at[pl.ds(step * block, block), :],
            out_sem.at[slot]).start()

    def wait_out(slot):
        pltpu.make_async_copy(
            o_buf.at[slot], o_hbm.at[pl.ds(0, block), :],
            out_sem.at[slot]).wait()

    dma_in(0, 0)
    w_bf[...] = w_ref[...].astype(jnp.bfloat16)

    def body(step, _):
        cur = jax.lax.rem(step, 2)
        nxt = jax.lax.rem(step + 1, 2)

        @pl.when(step + 1 < n_steps)
        def _():
            dma_in(nxt, step + 1)

        wait_in(cur)

        @pl.when(step >= 2)
        def _():
            wait_out(cur)

        x_bf[...] = x_buf[cur].astype(jnp.bfloat16)
        o_buf[cur] = _dot_bias(x_bf[...], w_bf[...], b_ref[...],
                               o_buf.dtype)
        dma_out(cur, step)
        return ()

    jax.lax.fori_loop(0, n_steps, body, (), unroll=False)
    if n_steps >= 2:
        wait_out((n_steps - 2) % 2)
    wait_out((n_steps - 1) % 2)


def _auto_kernel(x_ref, w_ref, b_ref, o_ref):
    o_ref[...] = _dot_bias(x_ref[...], w_ref[...], b_ref[...], o_ref.dtype)


def _forward_auto(x, weight, b_row):
    n, k = x.shape
    h = weight.shape[0]
    tm = min(2048, n)
    return pl.pallas_call(
        _auto_kernel,
        out_shape=jax.ShapeDtypeStruct((n, h), x.dtype),
        grid=(pl.cdiv(n, tm),),
        in_specs=[
            pl.BlockSpec((tm, k), lambda i: (i, 0)),
            pl.BlockSpec((h, k), lambda i: (0, 0)),
            pl.BlockSpec((1, h), lambda i: (0, 0)),
        ],
        out_specs=pl.BlockSpec((tm, h), lambda i: (i, 0)),
        compiler_params=pltpu.CompilerParams(
            dimension_semantics=("arbitrary",),
            vmem_limit_bytes=_VMEM_BUDGET,
        ),
    )(x, weight, b_row)


def kernel(x, weight, bias):
    n, k = x.shape
    h = weight.shape[0]
    b_row = bias.reshape(1, h)

    if n % _BLOCK != 0:
        return _forward_auto(x, weight, b_row)

    n_steps = n // _BLOCK
    bytes_accessed = (x.size * 4 + weight.size * 4 + h * 4 + n * h * 4)

    return pl.pallas_call(
        functools.partial(_manual_kernel, block=_BLOCK, n_steps=n_steps),
        out_shape=jax.ShapeDtypeStruct((n, h), x.dtype),
        in_specs=[
            pl.BlockSpec(memory_space=pl.ANY),
            pl.BlockSpec(memory_space=pltpu.VMEM),
            pl.BlockSpec(memory_space=pltpu.VMEM),
        ],
        out_specs=pl.BlockSpec(memory_space=pl.ANY),
        scratch_shapes=[
            pltpu.VMEM((2, _BLOCK, k), x.dtype),
            pltpu.VMEM((2, _BLOCK, h), x.dtype),
            pltpu.VMEM((h, k), jnp.bfloat16),
            pltpu.VMEM((_BLOCK, k), jnp.bfloat16),
            pltpu.SemaphoreType.DMA((2,)),
            pltpu.SemaphoreType.DMA((2,)),
        ],
        compiler_params=pltpu.CompilerParams(
            vmem_limit_bytes=_VMEM_BUDGET,
        ),
        cost_estimate=pl.CostEstimate(
            flops=2 * n * h * k,
            bytes_accessed=bytes_accessed,
            transcendentals=0),
    )(x, weight, b_row)


# speedup vs baseline: 1.0108x; 1.0070x over previous
import functools

import jax
import jax.numpy as jnp
from jax.experimental import pallas as pl
from jax.experimental.pallas import tpu as pltpu

_VMEM_BUDGET = (64 * 1024 * 1024 * 3) // 4
_BLOCK = 1024


def _dot_bias(x, w, b, out_dtype):
    acc = jax.lax.dot_general(
        x, w, dimension_numbers=(((1,), (0,)), ((), ())),
        preferred_element_type=jnp.float32)
    return (acc + b.astype(jnp.float32)).astype(out_dtype)


def _manual_kernel(x_hbm, w_ref, b_ref, o_hbm, x_buf, o_buf, w_bf, x_bf,
                   in_sem, out_sem, *, block, n_steps):
    def dma_in(slot, step):
        pltpu.make_async_copy(
            x_hbm.at[pl.ds(step * block, block), :], x_buf.at[slot],
            in_sem.at[slot]).start()

    def wait_in(slot):
        pltpu.make_async_copy(
            x_hbm.at[pl.ds(0, block), :], x_buf.at[slot],
            in_sem.at[slot]).wait()

    def dma_out(slot, step):
        pltpu.make_async_copy(
            o_buf.at[slot], o_hbm.at[pl.ds(step * block, block), :],
            out_sem.at[slot]).start()

    def wait_out(slot):
        pltpu.make_async_copy(
            o_buf.at[slot], o_hbm.at[pl.ds(0, block), :],
            out_sem.at[slot]).wait()

    dma_in(0, 0)
    w_bf[...] = w_ref[...].astype(jnp.bfloat16).T

    def body(step, _):
        cur = jax.lax.rem(step, 2)
        nxt = jax.lax.rem(step + 1, 2)

        @pl.when(step + 1 < n_steps)
        def _():
            dma_in(nxt, step + 1)

        wait_in(cur)

        @pl.when(step >= 2)
        def _():
            wait_out(cur)

        x_bf[...] = x_buf[cur].astype(jnp.bfloat16)
        o_buf[cur] = _dot_bias(x_bf[...], w_bf[...], b_ref[...],
                               o_buf.dtype)
        dma_out(cur, step)
        return ()

    jax.lax.fori_loop(0, n_steps, body, (), unroll=False)
    if n_steps >= 2:
        wait_out((n_steps - 2) % 2)
    wait_out((n_steps - 1) % 2)


def _auto_kernel(x_ref, w_ref, b_ref, o_ref):
    acc = jax.lax.dot_general(
        x_ref[...], w_ref[...], dimension_numbers=(((1,), (1,)), ((), ())),
        preferred_element_type=jnp.float32)
    o_ref[...] = (acc + b_ref[...].astype(jnp.float32)).astype(o_ref.dtype)


def _forward_auto(x, weight, b_row):
    n, k = x.shape
    h = weight.shape[0]
    tm = min(2048, n)
    return pl.pallas_call(
        _auto_kernel,
        out_shape=jax.ShapeDtypeStruct((n, h), x.dtype),
        grid=(pl.cdiv(n, tm),),
        in_specs=[
            pl.BlockSpec((tm, k), lambda i: (i, 0)),
            pl.BlockSpec((h, k), lambda i: (0, 0)),
            pl.BlockSpec((1, h), lambda i: (0, 0)),
        ],
        out_specs=pl.BlockSpec((tm, h), lambda i: (i, 0)),
        compiler_params=pltpu.CompilerParams(
            dimension_semantics=("arbitrary",),
            vmem_limit_bytes=_VMEM_BUDGET,
        ),
    )(x, weight, b_row)


def kernel(x, weight, bias):
    n, k = x.shape
    h = weight.shape[0]
    b_row = bias.reshape(1, h)

    if n % _BLOCK != 0:
        return _forward_auto(x, weight, b_row)

    n_steps = n // _BLOCK
    bytes_accessed = (x.size * 4 + weight.size * 4 + h * 4 + n * h * 4)

    return pl.pallas_call(
        functools.partial(_manual_kernel, block=_BLOCK, n_steps=n_steps),
        out_shape=jax.ShapeDtypeStruct((n, h), x.dtype),
        in_specs=[
            pl.BlockSpec(memory_space=pl.ANY),
            pl.BlockSpec(memory_space=pltpu.VMEM),
            pl.BlockSpec(memory_space=pltpu.VMEM),
        ],
        out_specs=pl.BlockSpec(memory_space=pl.ANY),
        scratch_shapes=[
            pltpu.VMEM((2, _BLOCK, k), x.dtype),
            pltpu.VMEM((2, _BLOCK, h), x.dtype),
            pltpu.VMEM((k, h), jnp.bfloat16),
            pltpu.VMEM((_BLOCK, k), jnp.bfloat16),
            pltpu.SemaphoreType.DMA((2,)),
            pltpu.SemaphoreType.DMA((2,)),
        ],
        compiler_params=pltpu.CompilerParams(
            vmem_limit_bytes=_VMEM_BUDGET,
        ),
        cost_estimate=pl.CostEstimate(
            flops=2 * n * h * k,
            bytes_accessed=bytes_accessed,
            transcendentals=0),
    )(x, weight, b_row)


# speedup vs baseline: 1.0108x; 1.0000x over previous
import functools

import jax
import jax.numpy as jnp
from jax.experimental import pallas as pl
from jax.experimental.pallas import tpu as pltpu

_VMEM_BUDGET = (64 * 1024 * 1024 * 3) // 4
_BLOCK = 1024


def _manual_kernel(x_hbm, w_hbm, b_ref, o_hbm, x_buf, o_buf, w_f32, w_bf,
                   in_sem, out_sem, w_sem, *, block, n_steps):
    def dma_in(slot, step):
        pltpu.make_async_copy(
            x_hbm.at[pl.ds(step * block, block), :], x_buf.at[slot],
            in_sem.at[slot]).start()

    def wait_in(slot):
        pltpu.make_async_copy(
            x_hbm.at[pl.ds(0, block), :], x_buf.at[slot],
            in_sem.at[slot]).wait()

    def dma_out(slot, step):
        pltpu.make_async_copy(
            o_buf.at[slot], o_hbm.at[pl.ds(step * block, block), :],
            out_sem.at[slot]).start()

    def wait_out(slot):
        pltpu.make_async_copy(
            o_buf.at[slot], o_hbm.at[pl.ds(0, block), :],
            out_sem.at[slot]).wait()

    dma_in(0, 0)
    w_copy = pltpu.make_async_copy(w_hbm, w_f32, w_sem)
    w_copy.start()
    w_copy.wait()
    w_bf[...] = w_f32[...].astype(jnp.bfloat16).T

    def body(step, _):
        cur = jax.lax.rem(step, 2)
        nxt = jax.lax.rem(step + 1, 2)

        @pl.when(step + 1 < n_steps)
        def _():
            dma_in(nxt, step + 1)

        wait_in(cur)

        @pl.when(step >= 2)
        def _():
            wait_out(cur)

        acc = jax.lax.dot_general(
            x_buf[cur], w_bf[...],
            dimension_numbers=(((1,), (0,)), ((), ())),
            preferred_element_type=jnp.float32)
        o_buf[cur] = acc + b_ref[...]
        dma_out(cur, step)
        return ()

    jax.lax.fori_loop(0, n_steps, body, (), unroll=False)
    if n_steps >= 2:
        wait_out((n_steps - 2) % 2)
    wait_out((n_steps - 1) % 2)


def _auto_kernel(x_ref, w_ref, b_ref, o_ref):
    acc = jax.lax.dot_general(
        x_ref[...], w_ref[...], dimension_numbers=(((1,), (1,)), ((), ())),
        preferred_element_type=jnp.float32)
    o_ref[...] = (acc + b_ref[...].astype(jnp.float32)).astype(o_ref.dtype)


def _forward_auto(x, weight, b_row):
    n, k = x.shape
    h = weight.shape[0]
    tm = min(2048, n)
    return pl.pallas_call(
        _auto_kernel,
        out_shape=jax.ShapeDtypeStruct((n, h), x.dtype),
        grid=(pl.cdiv(n, tm),),
        in_specs=[
            pl.BlockSpec((tm, k), lambda i: (i, 0)),
            pl.BlockSpec((h, k), lambda i: (0, 0)),
            pl.BlockSpec((1, h), lambda i: (0, 0)),
        ],
        out_specs=pl.BlockSpec((tm, h), lambda i: (i, 0)),
        compiler_params=pltpu.CompilerParams(
            dimension_semantics=("arbitrary",),
            vmem_limit_bytes=_VMEM_BUDGET,
        ),
    )(x, weight, b_row)


def kernel(x, weight, bias):
    n, k = x.shape
    h = weight.shape[0]
    b_row = bias.reshape(1, h)

    if n % _BLOCK != 0 or x.dtype != jnp.float32:
        return _forward_auto(x, weight, b_row)

    n_steps = n // _BLOCK

    return pl.pallas_call(
        functools.partial(_manual_kernel, block=_BLOCK, n_steps=n_steps),
        out_shape=jax.ShapeDtypeStruct((n, h), x.dtype),
        in_specs=[
            pl.BlockSpec(memory_space=pl.ANY),
            pl.BlockSpec(memory_space=pl.ANY),
            pl.BlockSpec(memory_space=pltpu.VMEM),
        ],
        out_specs=pl.BlockSpec(memory_space=pl.ANY),
        scratch_shapes=[
            pltpu.VMEM((2, _BLOCK, k), x.dtype),
            pltpu.VMEM((2, _BLOCK, h), x.dtype),
            pltpu.VMEM((h, k), jnp.float32),
            pltpu.VMEM((k, h), jnp.bfloat16),
            pltpu.SemaphoreType.DMA((2,)),
            pltpu.SemaphoreType.DMA((2,)),
            pltpu.SemaphoreType.DMA,
        ],
        compiler_params=pltpu.CompilerParams(
            vmem_limit_bytes=_VMEM_BUDGET,
        ),
        cost_estimate=pl.CostEstimate(
            flops=2 * n * h * k,
            bytes_accessed=(x.size + weight.size + h + n * h) * 4,
            transcendentals=0),
    )(x, weight, b_row)
